# Initial kernel scaffold; baseline (speedup 1.0000x reference)
import jax
import jax.numpy as jnp
from jax.experimental import pallas as pl


def kernel(input_ids, weight):
    raise NotImplementedError("write your pallas kernel here")


# speedup vs baseline: 4.2255x; 4.2255x over previous
import functools

import jax
import jax.numpy as jnp
from jax import lax
from jax.experimental import pallas as pl
from jax.experimental.pallas import tpu as pltpu
from jax.experimental.pallas import tpu_sc as plsc

_NC = 2
_NS = 16
_NW = _NC * _NS


def _emb_call(n, D, CH, n_ch, b_per_w, idx, weight):
    mesh = plsc.VectorSubcoreMesh(core_axis_name="c", subcore_axis_name="s")

    @functools.partial(
        pl.kernel,
        out_type=jax.ShapeDtypeStruct((n, D), jnp.float32),
        mesh=mesh,
        scratch_types=[
            pltpu.VMEM((CH,), jnp.int32),
            pltpu.VMEM((CH, D), jnp.float32),
            pltpu.SemaphoreType.DMA,
        ],
        compiler_params=pltpu.CompilerParams(use_tc_tiling_on_sc=False),
    )
    def emb(idx_hbm, table_hbm, out_hbm, idx_v, rows_v, sem):
        wid = lax.axis_index("s") * _NC + lax.axis_index("c")
        base = wid * b_per_w

        def body(i, carry):
            off = base + i * CH
            pltpu.sync_copy(idx_hbm.at[pl.ds(off, CH)], idx_v)
            pltpu.async_copy(table_hbm.at[idx_v], rows_v, sem).wait()
            pltpu.sync_copy(rows_v, out_hbm.at[pl.ds(off, CH)])
            return carry

        lax.fori_loop(0, n_ch, body, 0)

    return emb(idx, weight)


def kernel(input_ids, weight):
    B, S = input_ids.shape
    V, D = weight.shape
    n = B * S
    idx = input_ids.reshape(n).astype(jnp.int32)

    b_per_w = n // _NW
    CH = 1600
    n_ch = b_per_w // CH

    out = _emb_call(n, D, CH, n_ch, b_per_w, idx, weight)
    return out.reshape(B, S, D)


# speedup vs baseline: 4.2542x; 1.0068x over previous
import functools

import jax
import jax.numpy as jnp
from jax import lax
from jax.experimental import pallas as pl
from jax.experimental.pallas import tpu as pltpu
from jax.experimental.pallas import tpu_sc as plsc

_NC = 2
_NS = 16
_NW = _NC * _NS
_NBUF = 2


def _emb_call(n, D, CH, n_ch, b_per_w, idx, weight):
    mesh = plsc.VectorSubcoreMesh(core_axis_name="c", subcore_axis_name="s")

    @functools.partial(
        pl.kernel,
        out_type=jax.ShapeDtypeStruct((n, D), jnp.float32),
        mesh=mesh,
        scratch_types=[
            pltpu.VMEM((b_per_w,), jnp.int32),
            [pltpu.VMEM((CH, D), jnp.float32) for _ in range(_NBUF)],
            [pltpu.SemaphoreType.DMA for _ in range(_NBUF)],
        ],
        compiler_params=pltpu.CompilerParams(use_tc_tiling_on_sc=False),
    )
    def emb(idx_hbm, table_hbm, out_hbm, idx_v, rows, gsem):
        wid = lax.axis_index("s") * _NC + lax.axis_index("c")
        base = wid * b_per_w

        pltpu.sync_copy(idx_hbm.at[pl.ds(base, b_per_w)], idx_v)
        for b in range(_NBUF):
            pltpu.async_copy(
                table_hbm.at[idx_v.at[pl.ds(b * CH, CH)]], rows[b], gsem[b])

        def outer(jo, carry):
            i0 = jo * _NBUF
            for b in range(_NBUF):
                i = i0 + b
                off = base + i * CH
                pltpu.make_async_copy(
                    table_hbm.at[pl.ds(0, CH)], rows[b], gsem[b]).wait()
                pltpu.sync_copy(rows[b], out_hbm.at[pl.ds(off, CH)])

                @pl.when(i + _NBUF < n_ch)
                def _():
                    nxt = i + _NBUF
                    pltpu.async_copy(
                        table_hbm.at[idx_v.at[pl.ds(nxt * CH, CH)]],
                        rows[b], gsem[b])
            return carry

        lax.fori_loop(0, n_ch // _NBUF, outer, 0)

    return emb(idx, weight)


def kernel(input_ids, weight):
    B, S = input_ids.shape
    V, D = weight.shape
    n = B * S
    idx = input_ids.reshape(n).astype(jnp.int32)

    b_per_w = n // _NW
    CH = 800
    n_ch = b_per_w // CH

    out = _emb_call(n, D, CH, n_ch, b_per_w, idx, weight)
    return out.reshape(B, S, D)


# speedup vs baseline: 5.5844x; 1.3127x over previous
import functools

import jax
import jax.numpy as jnp
from jax import lax
from jax.experimental import pallas as pl
from jax.experimental.pallas import tpu as pltpu
from jax.experimental.pallas import tpu_sc as plsc

_NC = 2
_NS = 16
_NW = _NC * _NS
_NBUF = 4


def _emb_call(B, S, DP, s_per_w, idx, weight_p):
    mesh = plsc.VectorSubcoreMesh(core_axis_name="c", subcore_axis_name="s")
    n_per_w = s_per_w * S

    @functools.partial(
        pl.kernel,
        out_type=jax.ShapeDtypeStruct((B, S, DP), jnp.float32),
        mesh=mesh,
        scratch_types=[
            pltpu.VMEM((n_per_w,), jnp.int32),
            [pltpu.VMEM((S, DP), jnp.float32) for _ in range(_NBUF)],
            [pltpu.SemaphoreType.DMA for _ in range(_NBUF)],
        ],
    )
    def emb(idx_hbm, table_hbm, out_hbm, idx_v, rows, gsem):
        wid = lax.axis_index("s") * _NC + lax.axis_index("c")
        seq_base = wid * s_per_w

        pltpu.sync_copy(idx_hbm.at[pl.ds(seq_base * S, n_per_w)], idx_v)
        for b in range(_NBUF):
            pltpu.async_copy(
                table_hbm.at[idx_v.at[pl.ds(b * S, S)]], rows[b], gsem[b])

        def outer(jo, carry):
            i0 = jo * _NBUF
            for b in range(_NBUF):
                i = i0 + b
                pltpu.make_async_copy(
                    table_hbm.at[pl.ds(0, S)], rows[b], gsem[b]).wait()
                pltpu.sync_copy(rows[b], out_hbm.at[seq_base + i])

                @pl.when(i + _NBUF < s_per_w)
                def _():
                    nxt = i + _NBUF
                    pltpu.async_copy(
                        table_hbm.at[idx_v.at[pl.ds(nxt * S, S)]],
                        rows[b], gsem[b])
            return carry

        lax.fori_loop(0, s_per_w // _NBUF, outer, 0)

    return emb(idx, weight_p)


def kernel(input_ids, weight):
    B, S = input_ids.shape
    V, D = weight.shape
    DP = 128
    idx = input_ids.reshape(B * S).astype(jnp.int32)
    weight_p = jnp.pad(weight, ((0, 0), (0, DP - D)))

    s_per_w = B // _NW

    out = _emb_call(B, S, DP, s_per_w, idx, weight_p)
    return out[:, :, :D]
